# baseline (speedup 1.0000x reference)
import functools

import jax
import jax.numpy as jnp
from jax import lax
from jax.experimental import pallas as pl
from jax.experimental.pallas import tpu as pltpu
from jax.experimental.pallas import tpu_sc as plsc

B, P, D, K = 2, 4096, 768, 16
NC, NS, L = 2, 16, 16
ROWS = B * P
RPW = P // NS
DW = D // 2
DV = DW // L
G = 8
NBUF = 2
NG = RPW // G
DU = 8
PR = 32
NPC = RPW // PR

_mesh = plsc.VectorSubcoreMesh(
    core_axis_name="c", subcore_axis_name="s", num_cores=NC, num_subcores=NS
)
_params = pltpu.CompilerParams(needs_layout_passes=False)


def _rtne_bf16_bits(u):
    return (u + 0x7FFF + ((u >> 16) & 1)) >> 16


@functools.partial(
    pl.kernel,
    out_type=(
        jax.ShapeDtypeStruct((ROWS, D), jnp.float32),
        jax.ShapeDtypeStruct((ROWS, DW), jnp.uint32),
    ),
    mesh=_mesh,
    compiler_params=_params,
    scratch_types=[
        pltpu.VMEM((RPW * K,), jnp.int32),
        pltpu.VMEM((RPW * K,), jnp.float32),
        pltpu.SemaphoreType.DMA,
        pltpu.SemaphoreType.DMA,
    ],
)
def _wormhole_gather(x_hbm, routes_hbm, weights_hbm, out_hbm, xp_hbm,
                     idx_all, w_all, isem, wsem):
    c = lax.axis_index("c")
    s = lax.axis_index("s")
    xbase = c * P + s * RPW

    base_all = pl.multiple_of(xbase * K, RPW * K)
    pltpu.make_async_copy(
        routes_hbm.at[pl.ds(base_all, RPW * K)], idx_all, isem
    ).start()
    pltpu.make_async_copy(
        weights_hbm.at[pl.ds(base_all, RPW * K)], w_all, wsem
    ).start()

    def pack_phase(in_bufs, pk_bufs, psems):
        def pk_issue(pc, slot):
            pltpu.make_async_copy(
                x_hbm.at[pl.ds(xbase + pc * PR, PR)], in_bufs[slot],
                psems[slot],
            ).start()

        def pk_convert(pc, slot):
            pltpu.make_async_copy(
                x_hbm.at[pl.ds(xbase + pc * PR, PR)], in_bufs[slot],
                psems[slot],
            ).wait()

            def rbody(r, carry, slot=slot):
                for j in range(DV):
                    a = plsc.bitcast(in_bufs[slot][r, pl.ds(j * L, L)],
                                     jnp.uint32)
                    b = plsc.bitcast(
                        in_bufs[slot][r, pl.ds(DW + j * L, L)], jnp.uint32)
                    lo = _rtne_bf16_bits(a)
                    hi = _rtne_bf16_bits(b)
                    pk_bufs[slot][r, pl.ds(j * L, L)] = lo | (hi << 16)
                return carry

            lax.fori_loop(0, PR, rbody, 0)
            pltpu.sync_copy(pk_bufs[slot],
                            xp_hbm.at[pl.ds(xbase + pc * PR, PR)])

        pk_issue(0, 0)

        def pk_outer(cc, carry):
            for bslot in range(2):
                pc = cc * 2 + bslot

                @pl.when(pc + 1 < NPC)
                def _issue_next():
                    pk_issue(pc + 1, (bslot + 1) % 2)

                pk_convert(pc, bslot)
            return carry

        lax.fori_loop(0, NPC // 2, pk_outer, 0)

    pl.run_scoped(
        pack_phase,
        [pltpu.VMEM((PR, D), jnp.float32) for _ in range(2)],
        [pltpu.VMEM((PR, DW), jnp.uint32) for _ in range(2)],
        [pltpu.SemaphoreType.DMA for _ in range(2)],
    )

    plsc.subcore_barrier()
    pltpu.make_async_copy(
        routes_hbm.at[pl.ds(base_all, RPW * K)], idx_all, isem
    ).wait()
    pltpu.make_async_copy(
        weights_hbm.at[pl.ds(base_all, RPW * K)], w_all, wsem
    ).wait()
    boff = c * P
    off_splat = jnp.broadcast_to(boff, (L,)).astype(jnp.int32)

    def obody(i, carry):
        sl = pl.ds(i * L, L)
        idx_all[sl] = idx_all[sl] + off_splat
        return carry

    lax.fori_loop(0, RPW * K // L, obody, 0)

    def gather_phase(row_bufs, out_v, sems):
        def issue(gc, slot):
            pltpu.make_async_copy(
                xp_hbm.at[idx_all.at[pl.ds(gc * (G * K), G * K)]],
                row_bufs[slot], sems[slot],
            ).start()

        def compute(gc, slot):
            pltpu.make_async_copy(
                xp_hbm.at[idx_all.at[pl.ds(gc * (G * K), G * K)]],
                row_bufs[slot], sems[slot],
            ).wait()
            rows = row_bufs[slot]
            for g in range(G):
                wv = w_all[pl.ds((gc * G + g) * K, K)]
                r = _rtne_bf16_bits(plsc.bitcast(wv, jnp.uint32))
                pk = r | (r << 16)
                ws = [
                    plsc.bitcast(jnp.broadcast_to(pk[k], (L,)),
                                 jnp.bfloat16)
                    for k in range(K)
                ]

                def dbody(d, carry, g=g, ws=ws):
                    for u_ in range(DU):
                        j = d * DU + u_
                        sl = pl.ds(j * L, L)
                        ld = lambda k: plsc.bitcast(rows[g * K + k, sl],
                                                    jnp.bfloat16)
                        acc = [ld(a) * ws[a] for a in range(4)]
                        for k in range(4, K):
                            acc[k % 4] = acc[k % 4] + ld(k) * ws[k]
                        ssum = (acc[0] + acc[1]) + (acc[2] + acc[3])
                        su = plsc.bitcast(ssum, jnp.uint32)
                        out_v[g, sl] = plsc.bitcast(su << 16, jnp.float32)
                        out_v[g, pl.ds(DW + j * L, L)] = plsc.bitcast(
                            su & jnp.uint32(0xFFFF0000), jnp.float32)
                    return carry

                lax.fori_loop(0, DV // DU, dbody, 0)
            pltpu.sync_copy(out_v, out_hbm.at[pl.ds(xbase + gc * G, G)])

        issue(0, 0)

        def outer(cc, carry):
            for bslot in range(NBUF):
                gc = cc * NBUF + bslot

                @pl.when(gc + 1 < NG)
                def _issue_next():
                    issue(gc + 1, (bslot + 1) % NBUF)

                compute(gc, bslot)
            return carry

        lax.fori_loop(0, NG // NBUF, outer, 0)

    pl.run_scoped(
        gather_phase,
        [pltpu.VMEM((G * K, DW), jnp.uint32) for _ in range(NBUF)],
        pltpu.VMEM((G, D), jnp.float32),
        [pltpu.SemaphoreType.DMA for _ in range(NBUF)],
    )


def kernel(x, routes, weights):
    x_flat = x.reshape(ROWS, D)
    r_flat = routes.astype(jnp.int32).reshape(ROWS * K)
    w_flat = weights.reshape(ROWS * K)
    out, _ = _wormhole_gather(x_flat, r_flat, w_flat)
    return out.reshape(B, P, D)


# speedup vs baseline: 1.4163x; 1.4163x over previous
import functools

import jax
import jax.numpy as jnp
from jax import lax
from jax.experimental import pallas as pl
from jax.experimental.pallas import tpu as pltpu
from jax.experimental.pallas import tpu_sc as plsc

B, P, D, K = 2, 4096, 768, 16
NC, NS, L = 2, 16, 16
ROWS = B * P
RPW = P // NS
DW = D // 2
DV = DW // L
G = 8
NBUF = 2
NG = RPW // G
DU = 4
PR = 32
NPC = RPW // PR

_mesh = plsc.VectorSubcoreMesh(
    core_axis_name="c", subcore_axis_name="s", num_cores=NC, num_subcores=NS
)
_params = pltpu.CompilerParams(needs_layout_passes=False)


def _rtne_bf16_bits(u):
    return (u + 0x7FFF + ((u >> 16) & 1)) >> 16


@functools.partial(
    pl.kernel,
    out_type=(
        jax.ShapeDtypeStruct((ROWS, D), jnp.float32),
        jax.ShapeDtypeStruct((ROWS, DW), jnp.uint32),
    ),
    mesh=_mesh,
    compiler_params=_params,
    scratch_types=[
        pltpu.VMEM((RPW * K,), jnp.int32),
        pltpu.VMEM((RPW * K,), jnp.float32),
        pltpu.SemaphoreType.DMA,
        pltpu.SemaphoreType.DMA,
    ],
)
def _wormhole_gather(x_hbm, routes_hbm, weights_hbm, out_hbm, xp_hbm,
                     idx_all, w_all, isem, wsem):
    c = lax.axis_index("c")
    s = lax.axis_index("s")
    xbase = c * P + s * RPW

    base_all = pl.multiple_of(xbase * K, RPW * K)
    pltpu.make_async_copy(
        routes_hbm.at[pl.ds(base_all, RPW * K)], idx_all, isem
    ).start()
    pltpu.make_async_copy(
        weights_hbm.at[pl.ds(base_all, RPW * K)], w_all, wsem
    ).start()

    def pack_phase(in_bufs, pk_bufs, psems, osems):
        def pk_issue(pc, slot):
            pltpu.make_async_copy(
                x_hbm.at[pl.ds(xbase + pc * PR, PR)], in_bufs[slot],
                psems[slot],
            ).start()

        def pk_convert(pc, slot):
            pltpu.make_async_copy(
                x_hbm.at[pl.ds(xbase + pc * PR, PR)], in_bufs[slot],
                psems[slot],
            ).wait()

            @pl.when(pc >= 2)
            def _drain():
                pltpu.make_async_copy(
                    pk_bufs[slot],
                    xp_hbm.at[pl.ds(xbase + (pc - 2) * PR, PR)],
                    osems[slot],
                ).wait()

            def rbody(r, carry, slot=slot):
                for j in range(DV):
                    a = plsc.bitcast(in_bufs[slot][r, pl.ds(j * L, L)],
                                     jnp.uint32)
                    b = plsc.bitcast(
                        in_bufs[slot][r, pl.ds(DW + j * L, L)], jnp.uint32)
                    lo = _rtne_bf16_bits(a)
                    hi = _rtne_bf16_bits(b)
                    pk_bufs[slot][r, pl.ds(j * L, L)] = lo | (hi << 16)
                return carry

            lax.fori_loop(0, PR, rbody, 0)
            pltpu.make_async_copy(
                pk_bufs[slot], xp_hbm.at[pl.ds(xbase + pc * PR, PR)],
                osems[slot],
            ).start()

        pk_issue(0, 0)

        def pk_outer(cc, carry):
            for bslot in range(2):
                pc = cc * 2 + bslot

                @pl.when(pc + 1 < NPC)
                def _issue_next():
                    pk_issue(pc + 1, (bslot + 1) % 2)

                pk_convert(pc, bslot)
            return carry

        lax.fori_loop(0, NPC // 2, pk_outer, 0)
        for pc in (NPC - 2, NPC - 1):
            pltpu.make_async_copy(
                pk_bufs[pc % 2], xp_hbm.at[pl.ds(xbase + pc * PR, PR)],
                osems[pc % 2],
            ).wait()

    pl.run_scoped(
        pack_phase,
        [pltpu.VMEM((PR, D), jnp.float32) for _ in range(2)],
        [pltpu.VMEM((PR, DW), jnp.uint32) for _ in range(2)],
        [pltpu.SemaphoreType.DMA for _ in range(2)],
        [pltpu.SemaphoreType.DMA for _ in range(2)],
    )

    plsc.subcore_barrier()
    pltpu.make_async_copy(
        routes_hbm.at[pl.ds(base_all, RPW * K)], idx_all, isem
    ).wait()
    pltpu.make_async_copy(
        weights_hbm.at[pl.ds(base_all, RPW * K)], w_all, wsem
    ).wait()
    boff = c * P
    off_splat = jnp.broadcast_to(boff, (L,)).astype(jnp.int32)

    def obody(i, carry):
        sl = pl.ds(i * L, L)
        idx_all[sl] = idx_all[sl] + off_splat
        return carry

    lax.fori_loop(0, RPW * K // L, obody, 0)

    def gather_phase(row_bufs, out_bufs, sems, osems2):
        def issue(gc, slot):
            pltpu.make_async_copy(
                xp_hbm.at[idx_all.at[pl.ds(gc * (G * K), G * K)]],
                row_bufs[slot], sems[slot],
            ).start()

        def compute(gc, slot):
            pltpu.make_async_copy(
                xp_hbm.at[idx_all.at[pl.ds(gc * (G * K), G * K)]],
                row_bufs[slot], sems[slot],
            ).wait()
            rows = row_bufs[slot]
            out_v = out_bufs[slot]

            @pl.when(gc >= NBUF)
            def _drain():
                pltpu.make_async_copy(
                    out_v, out_hbm.at[pl.ds(xbase + (gc - NBUF) * G, G)],
                    osems2[slot],
                ).wait()

            for g in range(G):
                wv = w_all[pl.ds((gc * G + g) * K, K)]
                r = _rtne_bf16_bits(plsc.bitcast(wv, jnp.uint32))
                pk = r | (r << 16)
                ws = [
                    plsc.bitcast(jnp.broadcast_to(pk[k], (L,)),
                                 jnp.bfloat16)
                    for k in range(K)
                ]

                def dbody(d, carry, g=g, ws=ws):
                    for u_ in range(DU):
                        j = d * DU + u_
                        sl = pl.ds(j * L, L)
                        ld = lambda k: plsc.bitcast(rows[g * K + k, sl],
                                                    jnp.bfloat16)
                        acc = [ld(a) * ws[a] for a in range(4)]
                        for k in range(4, K):
                            acc[k % 4] = acc[k % 4] + ld(k) * ws[k]
                        ssum = (acc[0] + acc[1]) + (acc[2] + acc[3])
                        su = plsc.bitcast(ssum, jnp.uint32)
                        out_v[g, sl] = plsc.bitcast(su << 16, jnp.float32)
                        out_v[g, pl.ds(DW + j * L, L)] = plsc.bitcast(
                            su & jnp.uint32(0xFFFF0000), jnp.float32)
                    return carry

                lax.fori_loop(0, DV // DU, dbody, 0)
            pltpu.make_async_copy(
                out_v, out_hbm.at[pl.ds(xbase + gc * G, G)], osems2[slot]
            ).start()

        issue(0, 0)

        def outer(cc, carry):
            for bslot in range(NBUF):
                gc = cc * NBUF + bslot

                @pl.when(gc + 1 < NG)
                def _issue_next():
                    issue(gc + 1, (bslot + 1) % NBUF)

                compute(gc, bslot)
            return carry

        lax.fori_loop(0, NG // NBUF, outer, 0)
        for gc in range(NG - NBUF, NG):
            pltpu.make_async_copy(
                out_bufs[gc % NBUF],
                out_hbm.at[pl.ds(xbase + gc * G, G)],
                osems2[gc % NBUF],
            ).wait()

    pl.run_scoped(
        gather_phase,
        [pltpu.VMEM((G * K, DW), jnp.uint32) for _ in range(NBUF)],
        [pltpu.VMEM((G, D), jnp.float32) for _ in range(NBUF)],
        [pltpu.SemaphoreType.DMA for _ in range(NBUF)],
        [pltpu.SemaphoreType.DMA for _ in range(NBUF)],
    )


def kernel(x, routes, weights):
    x_flat = x.reshape(ROWS, D)
    r_flat = routes.astype(jnp.int32).reshape(ROWS * K)
    w_flat = weights.reshape(ROWS * K)
    out, _ = _wormhole_gather(x_flat, r_flat, w_flat)
    return out.reshape(B, P, D)


# speedup vs baseline: 1.4363x; 1.0141x over previous
import functools

import jax
import jax.numpy as jnp
from jax import lax
from jax.experimental import pallas as pl
from jax.experimental.pallas import tpu as pltpu
from jax.experimental.pallas import tpu_sc as plsc

B, P, D, K = 2, 4096, 768, 16
NC, NS, L = 2, 16, 16
ROWS = B * P
RPW = P // NS
DW = D // 2
DV = DW // L
G = 8
NBUF = 2
NG = RPW // G
DU = 4
PR = 32
NPC = RPW // PR

_mesh = plsc.VectorSubcoreMesh(
    core_axis_name="c", subcore_axis_name="s", num_cores=NC, num_subcores=NS
)
_params = pltpu.CompilerParams(needs_layout_passes=False)


def _rtne_bf16_bits(u):
    return (u + 0x7FFF + ((u >> 16) & 1)) >> 16


@functools.partial(
    pl.kernel,
    out_type=(
        jax.ShapeDtypeStruct((ROWS, D), jnp.float32),
        jax.ShapeDtypeStruct((ROWS, DW), jnp.uint32),
    ),
    mesh=_mesh,
    compiler_params=_params,
    scratch_types=[
        pltpu.VMEM((RPW * K,), jnp.int32),
        pltpu.VMEM((RPW * K,), jnp.float32),
        pltpu.SemaphoreType.DMA,
        pltpu.SemaphoreType.DMA,
    ],
)
def _wormhole_gather(x_hbm, routes_hbm, weights_hbm, out_hbm, xp_hbm,
                     idx_all, w_all, isem, wsem):
    c = lax.axis_index("c")
    s = lax.axis_index("s")
    xbase = c * P + s * RPW

    base_all = pl.multiple_of(xbase * K, RPW * K)
    pltpu.make_async_copy(
        routes_hbm.at[pl.ds(base_all, RPW * K)], idx_all, isem
    ).start()
    pltpu.make_async_copy(
        weights_hbm.at[pl.ds(base_all, RPW * K)], w_all, wsem
    ).start()

    def pack_phase(in_bufs, pk_bufs, psems, osems):
        def pk_issue(pc, slot):
            pltpu.make_async_copy(
                x_hbm.at[pl.ds(xbase + pc * PR, PR)], in_bufs[slot],
                psems[slot],
            ).start()

        def pk_convert(pc, slot):
            pltpu.make_async_copy(
                x_hbm.at[pl.ds(xbase + pc * PR, PR)], in_bufs[slot],
                psems[slot],
            ).wait()

            @pl.when(pc >= 2)
            def _drain():
                pltpu.make_async_copy(
                    pk_bufs[slot],
                    xp_hbm.at[pl.ds(xbase + (pc - 2) * PR, PR)],
                    osems[slot],
                ).wait()

            def rbody(rr, carry, slot=slot):
                for u_ in range(2):
                    r = rr * 2 + u_
                    for j in range(DV):
                        a = plsc.bitcast(in_bufs[slot][r, pl.ds(j * L, L)],
                                         jnp.uint32)
                        b = plsc.bitcast(
                            in_bufs[slot][r, pl.ds(DW + j * L, L)],
                            jnp.uint32)
                        lo = _rtne_bf16_bits(a)
                        hi = _rtne_bf16_bits(b)
                        pk_bufs[slot][r, pl.ds(j * L, L)] = lo | (hi << 16)
                return carry

            lax.fori_loop(0, PR // 2, rbody, 0)
            pltpu.make_async_copy(
                pk_bufs[slot], xp_hbm.at[pl.ds(xbase + pc * PR, PR)],
                osems[slot],
            ).start()

        pk_issue(0, 0)

        def pk_outer(cc, carry):
            for bslot in range(2):
                pc = cc * 2 + bslot

                @pl.when(pc + 1 < NPC)
                def _issue_next():
                    pk_issue(pc + 1, (bslot + 1) % 2)

                pk_convert(pc, bslot)
            return carry

        lax.fori_loop(0, NPC // 2, pk_outer, 0)
        for pc in (NPC - 2, NPC - 1):
            pltpu.make_async_copy(
                pk_bufs[pc % 2], xp_hbm.at[pl.ds(xbase + pc * PR, PR)],
                osems[pc % 2],
            ).wait()

    pl.run_scoped(
        pack_phase,
        [pltpu.VMEM((PR, D), jnp.float32) for _ in range(2)],
        [pltpu.VMEM((PR, DW), jnp.uint32) for _ in range(2)],
        [pltpu.SemaphoreType.DMA for _ in range(2)],
        [pltpu.SemaphoreType.DMA for _ in range(2)],
    )

    plsc.subcore_barrier()
    pltpu.make_async_copy(
        routes_hbm.at[pl.ds(base_all, RPW * K)], idx_all, isem
    ).wait()
    pltpu.make_async_copy(
        weights_hbm.at[pl.ds(base_all, RPW * K)], w_all, wsem
    ).wait()
    boff = c * P
    off_splat = jnp.broadcast_to(boff, (L,)).astype(jnp.int32)

    def obody(i, carry):
        sl = pl.ds(i * L, L)
        idx_all[sl] = idx_all[sl] + off_splat
        return carry

    lax.fori_loop(0, RPW * K // L, obody, 0)

    def gather_phase(row_bufs, out_bufs, sems, osems2):
        def issue(gc, slot):
            pltpu.make_async_copy(
                xp_hbm.at[idx_all.at[pl.ds(gc * (G * K), G * K)]],
                row_bufs[slot], sems[slot],
            ).start()

        def compute(gc, slot):
            pltpu.make_async_copy(
                xp_hbm.at[idx_all.at[pl.ds(gc * (G * K), G * K)]],
                row_bufs[slot], sems[slot],
            ).wait()
            rows = row_bufs[slot]
            out_v = out_bufs[slot]

            @pl.when(gc >= NBUF)
            def _drain():
                pltpu.make_async_copy(
                    out_v, out_hbm.at[pl.ds(xbase + (gc - NBUF) * G, G)],
                    osems2[slot],
                ).wait()

            for g in range(G):
                wv = w_all[pl.ds((gc * G + g) * K, K)]
                r = _rtne_bf16_bits(plsc.bitcast(wv, jnp.uint32))
                pk = r | (r << 16)
                ws = [
                    plsc.bitcast(jnp.broadcast_to(pk[k], (L,)),
                                 jnp.bfloat16)
                    for k in range(K)
                ]

                def dbody(d, carry, g=g, ws=ws):
                    for u_ in range(DU):
                        j = d * DU + u_
                        sl = pl.ds(j * L, L)
                        ld = lambda k: plsc.bitcast(rows[g * K + k, sl],
                                                    jnp.bfloat16)
                        acc = [ld(a) * ws[a] for a in range(4)]
                        for k in range(4, K):
                            acc[k % 4] = acc[k % 4] + ld(k) * ws[k]
                        ssum = (acc[0] + acc[1]) + (acc[2] + acc[3])
                        su = plsc.bitcast(ssum, jnp.uint32)
                        out_v[g, sl] = plsc.bitcast(su << 16, jnp.float32)
                        out_v[g, pl.ds(DW + j * L, L)] = plsc.bitcast(
                            su & jnp.uint32(0xFFFF0000), jnp.float32)
                    return carry

                lax.fori_loop(0, DV // DU, dbody, 0)
            pltpu.make_async_copy(
                out_v, out_hbm.at[pl.ds(xbase + gc * G, G)], osems2[slot]
            ).start()

        issue(0, 0)

        def outer(cc, carry):
            for bslot in range(NBUF):
                gc = cc * NBUF + bslot

                @pl.when(gc + 1 < NG)
                def _issue_next():
                    issue(gc + 1, (bslot + 1) % NBUF)

                compute(gc, bslot)
            return carry

        lax.fori_loop(0, NG // NBUF, outer, 0)
        for gc in range(NG - NBUF, NG):
            pltpu.make_async_copy(
                out_bufs[gc % NBUF],
                out_hbm.at[pl.ds(xbase + gc * G, G)],
                osems2[gc % NBUF],
            ).wait()

    pl.run_scoped(
        gather_phase,
        [pltpu.VMEM((G * K, DW), jnp.uint32) for _ in range(NBUF)],
        [pltpu.VMEM((G, D), jnp.float32) for _ in range(NBUF)],
        [pltpu.SemaphoreType.DMA for _ in range(NBUF)],
        [pltpu.SemaphoreType.DMA for _ in range(NBUF)],
    )


def kernel(x, routes, weights):
    x_flat = x.reshape(ROWS, D)
    r_flat = routes.astype(jnp.int32).reshape(ROWS * K)
    w_flat = weights.reshape(ROWS * K)
    out, _ = _wormhole_gather(x_flat, r_flat, w_flat)
    return out.reshape(B, P, D)
